# baseline (speedup 1.0000x reference)
import functools
import math

import jax
import jax.numpy as jnp
import numpy as np
from jax import lax
from jax.experimental import pallas as pl
from jax.experimental.pallas import tpu as pltpu
from jax.experimental.pallas import tpu_sc as plsc

SEQ_LEN = 24
TOP_K = 2
D_MODEL = 64
N_HEADS = 8
SEGMENT_SIZES = [24, 12, 8, 6, 4, 3, 2]
NUM_FREQS = SEQ_LEN // 2
SQRT2 = math.sqrt(2.0)
EPS = float(np.finfo(float).eps)
F32 = jnp.float32


def _gelu(y):
    return y * 0.5 * (1.0 + jax.lax.erf(y / SQRT2))


def _mmt(a, w):
    return jax.lax.dot_general(a, w, (((1,), (1,)), ((), ())),
                               preferred_element_type=F32)


def _sin_pe_np(n, d):
    pe = np.zeros((n, d), dtype=np.float32)
    pos = np.arange(n, dtype=np.float32)[:, None]
    div = np.exp(np.arange(0, d, 2, dtype=np.float32) * (-np.log(10000.0) / d))
    pe[:, 0::2] = np.sin(pos * div)
    pe[:, 1::2] = np.cos(pos * div)
    return pe


def _ln_gelu(y, g, bt):
    m = jnp.mean(y, axis=-1, keepdims=True)
    v = jnp.mean((y - m) ** 2, axis=-1, keepdims=True)
    yn = (y - m) * jax.lax.rsqrt(v + 1e-5) * g + bt
    return _gelu(yn)


def _s2d(y2, N, HH, C):
    x6 = y2.reshape(N, HH // 2, 2, HH // 2, 2, C)
    parts = [x6[:, :, 0, :, 0, :], x6[:, :, 0, :, 1, :],
             x6[:, :, 1, :, 0, :], x6[:, :, 1, :, 1, :]]
    p = jnp.concatenate(parts, axis=-1)
    return p.reshape(N * (HH // 2) * (HH // 2), 4 * C)


def _gate_body(x_ref, w1, b1, g1, t1, w2, b2, g2, t2, w3, b3, g3, t3,
               w4, b4, g4, t4, fw, fb, fre, fim, wg, out_ref):
    N = SEQ_LEN
    h = x_ref[...]
    y = _s2d(h, N, 16, D_MODEL)
    y = _ln_gelu(jnp.dot(y, w1[...], preferred_element_type=F32) + b1[...],
                 g1[...], t1[...])
    y = _s2d(y, N, 8, 128)
    y = _ln_gelu(jnp.dot(y, w2[...], preferred_element_type=F32) + b2[...],
                 g2[...], t2[...])
    y = _s2d(y, N, 4, 256)
    y = _ln_gelu(jnp.dot(y, w3[...], preferred_element_type=F32) + b3[...],
                 g3[...], t3[...])
    y = _s2d(y, N, 2, 512)
    y = _ln_gelu(jnp.dot(y, w4[...], preferred_element_type=F32) + b4[...],
                 g4[...], t4[...])
    h2 = _mmt(y, fw[...]) + fb[...]
    re = jnp.dot(fre[...], h2, preferred_element_type=F32)
    im = jnp.dot(fim[...], h2, preferred_element_type=F32)
    amp = jnp.sqrt(re * re + im * im).mean(axis=1).reshape(1, NUM_FREQS)
    logits = jnp.dot(amp, wg[...], preferred_element_type=F32)
    E = logits.shape[1]
    out_ref[...] = jnp.concatenate(
        [logits, jnp.full((1, 16 - E), -1e30, F32)], axis=1).reshape(1, 1, 16)


def _sc_bcast(v, idx):
    dn = lax.GatherDimensionNumbers(offset_dims=(), collapsed_slice_dims=(0,),
                                    start_index_map=(0,))
    return lax.gather(v, idx[:, None], dn, slice_sizes=(1,),
                      mode=lax.GatherScatterMode.PROMISE_IN_BOUNDS)


def _route(logits16):
    B = logits16.shape[0]
    info = plsc.get_sparse_core_info()
    NC = info.num_cores
    NW = NC * info.num_subcores
    mesh = plsc.VectorSubcoreMesh(core_axis_name="c", subcore_axis_name="s")
    lg_pad = jnp.concatenate(
        [logits16, jnp.full((NW - B, 16), -1e30, F32)], axis=0)

    @functools.partial(
        pl.kernel, mesh=mesh,
        out_type=[jax.ShapeDtypeStruct((NW, 16), F32),
                  jax.ShapeDtypeStruct((NW, 16), jnp.int32)],
        scratch_types=[pltpu.VMEM((16,), F32),
                       pltpu.VMEM((16,), F32),
                       pltpu.VMEM((16,), jnp.int32)],
        compiler_params=pltpu.CompilerParams(needs_layout_passes=False),
    )
    def route(lg_hbm, gates_hbm, idx_hbm, lg_v, gt_v, ix_v):
        wid = lax.axis_index("s") * NC + lax.axis_index("c")
        pltpu.sync_copy(lg_hbm.at[wid], lg_v)
        lv = lg_v[...]
        iota = lax.broadcasted_iota(jnp.int32, (16,), 0)
        ks, vs = plsc.sort_key_val(lv, iota, descending=True)
        zeros = iota * 0
        ones = zeros + 1
        m1v = _sc_bcast(ks, zeros)
        m2v = _sc_bcast(ks, ones)
        i1v = _sc_bcast(vs, zeros)
        i2v = _sc_bcast(vs, ones)
        rv = jnp.exp(m2v - m1v)
        gt = jnp.where(iota == i1v, 1.0,
                       jnp.where(iota == i2v, rv, 0.0)) / (1.0 + rv)
        gt_v[...] = gt
        ix_v[...] = jnp.where(iota == 0, i1v,
                              jnp.where(iota == 1, i2v, 0))
        pltpu.sync_copy(gt_v, gates_hbm.at[wid])
        pltpu.sync_copy(ix_v, idx_hbm.at[wid])

    gates32, idx32 = route(lg_pad)
    return gates32[:B], idx32[:B]


def _expert_body(*refs, B, HW, ND):
    idx_ref = refs[0]
    gates_ref = refs[1]
    x_ref = refs[2]
    acc_ref = refs[-1]
    o_ref = refs[-2]
    wrefs = refs[3:-2]
    b = pl.program_id(0)

    acc_ref[...] = jnp.zeros_like(acc_ref)

    xv = x_ref[...]
    h_in = jnp.concatenate(
        [xv[t * HW:(t + 1) * HW, :].T for t in range(SEQ_LEN)], axis=0)

    for kk in range(TOP_K):
      e = idx_ref[b, kk]
      g = gates_ref[b, e]
      for i, p in enumerate(SEGMENT_SIZES):
        n = SEQ_LEN // p
        ind = p * D_MODEL
        hid = (p // 2) * D_MODEL
        hd = hid // N_HEADS
        wq, wk, wv, wo, bo, peT = wrefs[6 * i:6 * i + 6]

        @pl.when((e == i) & (g > 0.0))
        def _(wq=wq, wk=wk, wv=wv, wo=wo, bo=bo, peT=peT,
              n=n, ind=ind, hid=hid, hd=hd, g=g):
            h = h_in
            pe = peT[...]
            qs, ks, vs = [], [], []
            for s in range(n):
                hs = h[s * ind:(s + 1) * ind, :]
                qs.append(jnp.dot(wq[...], hs, preferred_element_type=F32)
                          + pe[:, s:s + 1])
                ks.append(jnp.dot(wk[...], hs, preferred_element_type=F32)
                          + pe[:, n + s:n + s + 1])
                vs.append(jnp.dot(wv[...], hs, preferred_element_type=F32)
                          + pe[:, 2 * n + s:2 * n + s + 1])
            scale = 1.0 / math.sqrt(hd)
            vg = [v.reshape(N_HEADS, hd, HW) for v in vs]
            for s in range(n):
                sc = []
                for t in range(n):
                    prod = (qs[s] * ks[t]).reshape(N_HEADS, hd, HW)
                    sc.append(prod.sum(axis=1) * scale)
                m = sc[0]
                for t in range(1, n):
                    m = jnp.maximum(m, sc[t])
                ex = [jnp.exp(x - m) for x in sc]
                z = ex[0]
                for t in range(1, n):
                    z = z + ex[t]
                inv = 1.0 / z
                acc = (ex[0] * inv)[:, None, :] * vg[0]
                for t in range(1, n):
                    acc = acc + (ex[t] * inv)[:, None, :] * vg[t]
                out_s = acc.reshape(hid, HW)
                o_s = jnp.dot(wo[...], out_s,
                              preferred_element_type=F32) + bo[...]
                acc_ref[s * ind:(s + 1) * ind, :] += jnp.exp(o_s) * g

    c = acc_ref[...]
    c = jnp.log(jnp.where(c == 0.0, EPS, c))
    for t in range(SEQ_LEN):
        o_ref[t * HW:(t + 1) * HW, :] = c[t * D_MODEL:(t + 1) * D_MODEL, :].T


def kernel(x, params):
    B, T, H, W, D = x.shape
    HW = H * W
    ND = T * D
    nE = len(SEGMENT_SIZES)

    t = np.arange(SEQ_LEN, dtype=np.float64)
    f = np.arange(1, NUM_FREQS + 1, dtype=np.float64)
    ang = 2.0 * np.pi * f[:, None] * t[None, :] / SEQ_LEN
    scale = 1.0 / np.sqrt(SEQ_LEN)
    fre = jnp.asarray((np.cos(ang) * scale).astype(np.float32))
    fim = jnp.asarray((-np.sin(ang) * scale).astype(np.float32))
    gargs = [x.reshape(B * T * H * W, D)]
    gspecs = [pl.BlockSpec((T * H * W, D), lambda b: (b, 0))]
    gfull = lambda b: (0, 0)
    for c in params['convs']:
        cin = c['w'].shape[1]
        cout = c['w'].shape[0]
        gargs += [c['w'].transpose(2, 3, 1, 0).reshape(4 * cin, cout),
                  c['b'][None], c['g'][None], c['beta'][None]]
        gspecs += [pl.BlockSpec((4 * cin, cout), gfull),
                   pl.BlockSpec((1, cout), gfull),
                   pl.BlockSpec((1, cout), gfull),
                   pl.BlockSpec((1, cout), gfull)]
    gargs += [params['fuse_w'], params['fuse_b'][None], fre, fim,
              params['w_gate']]
    gspecs += [pl.BlockSpec(params['fuse_w'].shape, gfull),
               pl.BlockSpec((1, params['fuse_b'].shape[0]), gfull),
               pl.BlockSpec(fre.shape, gfull),
               pl.BlockSpec(fim.shape, gfull),
               pl.BlockSpec(params['w_gate'].shape, gfull)]
    logits16 = pl.pallas_call(
        _gate_body,
        grid=(B,),
        in_specs=gspecs,
        out_specs=pl.BlockSpec((1, 1, 16), lambda b: (b, 0, 0)),
        out_shape=jax.ShapeDtypeStruct((B, 1, 16), F32),
        compiler_params=pltpu.CompilerParams(
            dimension_semantics=("arbitrary",),
            vmem_limit_bytes=100 * 1024 * 1024),
    )(*gargs).reshape(B, 16)
    gates16, idx16 = _route(logits16)
    gates = gates16[:, :nE]
    idx2 = idx16[:, :TOP_K]

    xv = x.reshape(B * T * HW, D)

    in_specs = [
        pl.BlockSpec(memory_space=pltpu.SMEM),
        pl.BlockSpec((T * HW, D), lambda b, idx: (b, 0)),
    ]
    args = [gates, xv]
    full = lambda b, idx: (0, 0)
    for ei, p in enumerate(SEGMENT_SIZES):
        ep = params['experts'][ei]
        n = SEQ_LEN // p
        ind = p * D_MODEL
        hid = (p // 2) * D_MODEL
        pe = jnp.asarray(_sin_pe_np(n, ind))
        peT = jnp.concatenate(
            [jnp.dot(ep['wq'], pe.T), jnp.dot(ep['wk'], pe.T),
             jnp.dot(ep['wv'], pe.T)], axis=1)
        for a in (ep['wq'], ep['wk'], ep['wv'], ep['wo']):
            args.append(a)
            in_specs.append(pl.BlockSpec(a.shape, full))
        args.append(ep['bo'][:, None])
        in_specs.append(pl.BlockSpec((ind, 1), full))
        args.append(peT)
        in_specs.append(pl.BlockSpec((hid, 3 * n), full))

    out = pl.pallas_call(
        functools.partial(_expert_body, B=B, HW=HW, ND=ND),
        grid_spec=pltpu.PrefetchScalarGridSpec(
            num_scalar_prefetch=1,
            grid=(B,),
            in_specs=in_specs,
            out_specs=pl.BlockSpec((T * HW, D), lambda b, idx: (b, 0)),
            scratch_shapes=[pltpu.VMEM((ND, HW), F32)],
        ),
        out_shape=jax.ShapeDtypeStruct((B * T * HW, D), F32),
        compiler_params=pltpu.CompilerParams(
            dimension_semantics=("arbitrary",),
            vmem_limit_bytes=100 * 1024 * 1024,
        ),
    )(idx2, *args)

    return out.reshape(B, T, H, W, D)


# speedup vs baseline: 2.0539x; 2.0539x over previous
import functools
import math

import jax
import jax.numpy as jnp
import numpy as np
from jax import lax
from jax.experimental import pallas as pl
from jax.experimental.pallas import tpu as pltpu
from jax.experimental.pallas import tpu_sc as plsc

SEQ_LEN = 24
TOP_K = 2
D_MODEL = 64
N_HEADS = 8
SEGMENT_SIZES = [24, 12, 8, 6, 4, 3, 2]
NUM_FREQS = SEQ_LEN // 2
SQRT2 = math.sqrt(2.0)
EPS = float(np.finfo(float).eps)
F32 = jnp.float32


def _gelu(y):
    return y * 0.5 * (1.0 + jax.lax.erf(y / SQRT2))


def _mmt(a, w):
    return jax.lax.dot_general(a, w, (((1,), (1,)), ((), ())),
                               preferred_element_type=F32)


def _sin_pe_np(n, d):
    pe = np.zeros((n, d), dtype=np.float32)
    pos = np.arange(n, dtype=np.float32)[:, None]
    div = np.exp(np.arange(0, d, 2, dtype=np.float32) * (-np.log(10000.0) / d))
    pe[:, 0::2] = np.sin(pos * div)
    pe[:, 1::2] = np.cos(pos * div)
    return pe


def _ln_gelu(y, g, bt):
    m = jnp.mean(y, axis=-1, keepdims=True)
    v = jnp.mean((y - m) ** 2, axis=-1, keepdims=True)
    yn = (y - m) * jax.lax.rsqrt(v + 1e-5) * g + bt
    return _gelu(yn)


def _s2d(y2, N, HH, C):
    x6 = y2.reshape(N, HH // 2, 2, HH // 2, 2, C)
    parts = [x6[:, :, 0, :, 0, :], x6[:, :, 0, :, 1, :],
             x6[:, :, 1, :, 0, :], x6[:, :, 1, :, 1, :]]
    p = jnp.concatenate(parts, axis=-1)
    return p.reshape(N * (HH // 2) * (HH // 2), 4 * C)


def _gate_body(x_ref, w1, b1, g1, t1, w2, b2, g2, t2, w3, b3, g3, t3,
               w4, b4, g4, t4, fw, fb, fre, fim, wg, out_ref):
    N = SEQ_LEN
    h = x_ref[...]
    y = _s2d(h, N, 16, D_MODEL)
    y = _ln_gelu(jnp.dot(y, w1[...], preferred_element_type=F32) + b1[...],
                 g1[...], t1[...])
    y = _s2d(y, N, 8, 128)
    y = _ln_gelu(jnp.dot(y, w2[...], preferred_element_type=F32) + b2[...],
                 g2[...], t2[...])
    y = _s2d(y, N, 4, 256)
    y = _ln_gelu(jnp.dot(y, w3[...], preferred_element_type=F32) + b3[...],
                 g3[...], t3[...])
    y = _s2d(y, N, 2, 512)
    y = _ln_gelu(jnp.dot(y, w4[...], preferred_element_type=F32) + b4[...],
                 g4[...], t4[...])
    h2 = _mmt(y, fw[...]) + fb[...]
    re = jnp.dot(fre[...], h2, preferred_element_type=F32)
    im = jnp.dot(fim[...], h2, preferred_element_type=F32)
    amp = jnp.sqrt(re * re + im * im).mean(axis=1).reshape(1, NUM_FREQS)
    logits = jnp.dot(amp, wg[...], preferred_element_type=F32)
    E = logits.shape[1]
    out_ref[...] = jnp.concatenate(
        [logits, jnp.full((1, 16 - E), -1e30, F32)], axis=1).reshape(1, 1, 16)


def _sc_bcast(v, idx):
    dn = lax.GatherDimensionNumbers(offset_dims=(), collapsed_slice_dims=(0,),
                                    start_index_map=(0,))
    return lax.gather(v, idx[:, None], dn, slice_sizes=(1,),
                      mode=lax.GatherScatterMode.PROMISE_IN_BOUNDS)


def _route(logits16):
    B = logits16.shape[0]
    info = plsc.get_sparse_core_info()
    NC = info.num_cores
    NW = NC * info.num_subcores
    mesh = plsc.VectorSubcoreMesh(core_axis_name="c", subcore_axis_name="s")
    lg_pad = jnp.concatenate(
        [logits16, jnp.full((NW - B, 16), -1e30, F32)], axis=0)

    @functools.partial(
        pl.kernel, mesh=mesh,
        out_type=[jax.ShapeDtypeStruct((NW, 16), F32),
                  jax.ShapeDtypeStruct((NW, 16), jnp.int32)],
        scratch_types=[pltpu.VMEM((16,), F32),
                       pltpu.VMEM((16,), F32),
                       pltpu.VMEM((16,), jnp.int32)],
        compiler_params=pltpu.CompilerParams(needs_layout_passes=False),
    )
    def route(lg_hbm, gates_hbm, idx_hbm, lg_v, gt_v, ix_v):
        wid = lax.axis_index("s") * NC + lax.axis_index("c")
        pltpu.sync_copy(lg_hbm.at[wid], lg_v)
        lv = lg_v[...]
        iota = lax.broadcasted_iota(jnp.int32, (16,), 0)
        ks, vs = plsc.sort_key_val(lv, iota, descending=True)
        zeros = iota * 0
        ones = zeros + 1
        m1v = _sc_bcast(ks, zeros)
        m2v = _sc_bcast(ks, ones)
        i1v = _sc_bcast(vs, zeros)
        i2v = _sc_bcast(vs, ones)
        rv = jnp.exp(m2v - m1v)
        gt = jnp.where(iota == i1v, 1.0,
                       jnp.where(iota == i2v, rv, 0.0)) / (1.0 + rv)
        gt_v[...] = gt
        ix_v[...] = jnp.where(iota == 0, i1v,
                              jnp.where(iota == 1, i2v, 0))
        pltpu.sync_copy(gt_v, gates_hbm.at[wid])
        pltpu.sync_copy(ix_v, idx_hbm.at[wid])

    gates32, idx32 = route(lg_pad)
    return gates32[:B], idx32[:B]


def _expert_body(*refs, B, HW, ND):
    idx_ref = refs[0]
    gates_ref = refs[1]
    x_ref = refs[2]
    acc_ref = refs[-1]
    o_ref = refs[-2]
    wrefs = refs[3:-2]
    b = pl.program_id(0)
    kk = pl.program_id(1)
    e = idx_ref[b, kk]

    @pl.when(kk == 0)
    def _():
        acc_ref[...] = jnp.zeros_like(acc_ref)

    xv = x_ref[...]
    h_in = jnp.concatenate(
        [xv[t * HW:(t + 1) * HW, :].T for t in range(SEQ_LEN)], axis=0)

    g = gates_ref[b, e]
    for i, p in enumerate(SEGMENT_SIZES):
        n = SEQ_LEN // p
        ind = p * D_MODEL
        hid = (p // 2) * D_MODEL
        hd = hid // N_HEADS
        wq, wk, wv, wo, bo, peT = wrefs[6 * i:6 * i + 6]

        @pl.when((e == i) & (g > 0.0))
        def _(wq=wq, wk=wk, wv=wv, wo=wo, bo=bo, peT=peT,
              n=n, ind=ind, hid=hid, hd=hd):
            h = h_in
            pe = peT[...]
            qs, ks, vs = [], [], []
            for s in range(n):
                hs = h[s * ind:(s + 1) * ind, :]
                qs.append(jnp.dot(wq[...], hs, preferred_element_type=F32)
                          + pe[:, s:s + 1])
                ks.append(jnp.dot(wk[...], hs, preferred_element_type=F32)
                          + pe[:, n + s:n + s + 1])
                vs.append(jnp.dot(wv[...], hs, preferred_element_type=F32)
                          + pe[:, 2 * n + s:2 * n + s + 1])
            scale = 1.0 / math.sqrt(hd)
            vg = [v.reshape(N_HEADS, hd, HW) for v in vs]
            for s in range(n):
                sc = []
                for t in range(n):
                    prod = (qs[s] * ks[t]).reshape(N_HEADS, hd, HW)
                    sc.append(prod.sum(axis=1) * scale)
                m = sc[0]
                for t in range(1, n):
                    m = jnp.maximum(m, sc[t])
                ex = [jnp.exp(x - m) for x in sc]
                z = ex[0]
                for t in range(1, n):
                    z = z + ex[t]
                inv = 1.0 / z
                acc = (ex[0] * inv)[:, None, :] * vg[0]
                for t in range(1, n):
                    acc = acc + (ex[t] * inv)[:, None, :] * vg[t]
                out_s = acc.reshape(hid, HW)
                o_s = jnp.dot(wo[...], out_s,
                              preferred_element_type=F32) + bo[...]
                acc_ref[s * ind:(s + 1) * ind, :] += jnp.exp(o_s) * g

    @pl.when(kk == TOP_K - 1)
    def _():
        c = acc_ref[...]
        c = jnp.log(jnp.where(c == 0.0, EPS, c))
        for t in range(SEQ_LEN):
            o_ref[t * HW:(t + 1) * HW, :] = (
                c[t * D_MODEL:(t + 1) * D_MODEL, :].T)


def kernel(x, params):
    B, T, H, W, D = x.shape
    HW = H * W
    ND = T * D
    nE = len(SEGMENT_SIZES)

    t = np.arange(SEQ_LEN, dtype=np.float64)
    f = np.arange(1, NUM_FREQS + 1, dtype=np.float64)
    ang = 2.0 * np.pi * f[:, None] * t[None, :] / SEQ_LEN
    scale = 1.0 / np.sqrt(SEQ_LEN)
    fre = jnp.asarray((np.cos(ang) * scale).astype(np.float32))
    fim = jnp.asarray((-np.sin(ang) * scale).astype(np.float32))
    gargs = [x.reshape(B * T * H * W, D)]
    gspecs = [pl.BlockSpec((T * H * W, D), lambda b: (b, 0))]
    gfull = lambda b: (0, 0)
    for c in params['convs']:
        cin = c['w'].shape[1]
        cout = c['w'].shape[0]
        gargs += [c['w'].transpose(2, 3, 1, 0).reshape(4 * cin, cout),
                  c['b'][None], c['g'][None], c['beta'][None]]
        gspecs += [pl.BlockSpec((4 * cin, cout), gfull),
                   pl.BlockSpec((1, cout), gfull),
                   pl.BlockSpec((1, cout), gfull),
                   pl.BlockSpec((1, cout), gfull)]
    gargs += [params['fuse_w'], params['fuse_b'][None], fre, fim,
              params['w_gate']]
    gspecs += [pl.BlockSpec(params['fuse_w'].shape, gfull),
               pl.BlockSpec((1, params['fuse_b'].shape[0]), gfull),
               pl.BlockSpec(fre.shape, gfull),
               pl.BlockSpec(fim.shape, gfull),
               pl.BlockSpec(params['w_gate'].shape, gfull)]
    logits16 = pl.pallas_call(
        _gate_body,
        grid=(B,),
        in_specs=gspecs,
        out_specs=pl.BlockSpec((1, 1, 16), lambda b: (b, 0, 0)),
        out_shape=jax.ShapeDtypeStruct((B, 1, 16), F32),
        compiler_params=pltpu.CompilerParams(
            dimension_semantics=("arbitrary",),
            vmem_limit_bytes=100 * 1024 * 1024),
    )(*gargs).reshape(B, 16)
    gates16, idx16 = _route(logits16)
    gates = gates16[:, :nE]
    idx2 = idx16[:, :TOP_K]

    xv = x.reshape(B * T * HW, D)

    in_specs = [
        pl.BlockSpec(memory_space=pltpu.SMEM),
        pl.BlockSpec((T * HW, D), lambda b, k, idx: (b, 0)),
    ]
    args = [gates, xv]
    full = lambda b, k, idx: (0, 0)
    for ei, p in enumerate(SEGMENT_SIZES):
        ep = params['experts'][ei]
        n = SEQ_LEN // p
        ind = p * D_MODEL
        hid = (p // 2) * D_MODEL
        pe = jnp.asarray(_sin_pe_np(n, ind))
        peT = jnp.concatenate(
            [jnp.dot(ep['wq'], pe.T), jnp.dot(ep['wk'], pe.T),
             jnp.dot(ep['wv'], pe.T)], axis=1)
        for a in (ep['wq'], ep['wk'], ep['wv'], ep['wo']):
            args.append(a)
            in_specs.append(pl.BlockSpec(a.shape, full))
        args.append(ep['bo'][:, None])
        in_specs.append(pl.BlockSpec((ind, 1), full))
        args.append(peT)
        in_specs.append(pl.BlockSpec((hid, 3 * n), full))

    out = pl.pallas_call(
        functools.partial(_expert_body, B=B, HW=HW, ND=ND),
        grid_spec=pltpu.PrefetchScalarGridSpec(
            num_scalar_prefetch=1,
            grid=(B, TOP_K),
            in_specs=in_specs,
            out_specs=pl.BlockSpec((T * HW, D), lambda b, k, idx: (b, 0)),
            scratch_shapes=[pltpu.VMEM((ND, HW), F32)],
        ),
        out_shape=jax.ShapeDtypeStruct((B * T * HW, D), F32),
        compiler_params=pltpu.CompilerParams(
            dimension_semantics=("arbitrary", "arbitrary"),
            vmem_limit_bytes=100 * 1024 * 1024,
        ),
    )(idx2, *args)

    return out.reshape(B, T, H, W, D)


# speedup vs baseline: 2.1181x; 1.0312x over previous
import functools
import math

import jax
import jax.numpy as jnp
import numpy as np
from jax import lax
from jax.experimental import pallas as pl
from jax.experimental.pallas import tpu as pltpu
from jax.experimental.pallas import tpu_sc as plsc

SEQ_LEN = 24
TOP_K = 2
D_MODEL = 64
N_HEADS = 8
SEGMENT_SIZES = [24, 12, 8, 6, 4, 3, 2]
NUM_FREQS = SEQ_LEN // 2
SQRT2 = math.sqrt(2.0)
EPS = float(np.finfo(float).eps)
F32 = jnp.float32


def _gelu(y):
    return y * 0.5 * (1.0 + jax.lax.erf(y / SQRT2))


def _mmt(a, w):
    return jax.lax.dot_general(a, w, (((1,), (1,)), ((), ())),
                               preferred_element_type=F32)


def _sin_pe_np(n, d):
    pe = np.zeros((n, d), dtype=np.float32)
    pos = np.arange(n, dtype=np.float32)[:, None]
    div = np.exp(np.arange(0, d, 2, dtype=np.float32) * (-np.log(10000.0) / d))
    pe[:, 0::2] = np.sin(pos * div)
    pe[:, 1::2] = np.cos(pos * div)
    return pe


def _ln_gelu(y, g, bt):
    m = jnp.mean(y, axis=-1, keepdims=True)
    v = jnp.mean((y - m) ** 2, axis=-1, keepdims=True)
    yn = (y - m) * jax.lax.rsqrt(v + 1e-5) * g + bt
    return _gelu(yn)


def _s2d(y2, N, HH, C):
    x6 = y2.reshape(N, HH // 2, 2, HH // 2, 2, C)
    parts = [x6[:, :, 0, :, 0, :], x6[:, :, 0, :, 1, :],
             x6[:, :, 1, :, 0, :], x6[:, :, 1, :, 1, :]]
    p = jnp.concatenate(parts, axis=-1)
    return p.reshape(N * (HH // 2) * (HH // 2), 4 * C)


def _gate_body(x_ref, w1, b1, g1, t1, w2, b2, g2, t2, w3, b3, g3, t3,
               w4, b4, g4, t4, fw, fb, fre, fim, wg, out_ref, SB):
    N = SB * SEQ_LEN
    h = x_ref[...]
    y = _s2d(h, N, 16, D_MODEL)
    y = _ln_gelu(jnp.dot(y, w1[...], preferred_element_type=F32) + b1[...],
                 g1[...], t1[...])
    y = _s2d(y, N, 8, 128)
    y = _ln_gelu(jnp.dot(y, w2[...], preferred_element_type=F32) + b2[...],
                 g2[...], t2[...])
    y = _s2d(y, N, 4, 256)
    y = _ln_gelu(jnp.dot(y, w3[...], preferred_element_type=F32) + b3[...],
                 g3[...], t3[...])
    y = _s2d(y, N, 2, 512)
    y = _ln_gelu(jnp.dot(y, w4[...], preferred_element_type=F32) + b4[...],
                 g4[...], t4[...])
    h2 = _mmt(y, fw[...]) + fb[...]
    rows = []
    for bb in range(SB):
        hb = h2[bb * SEQ_LEN:(bb + 1) * SEQ_LEN, :]
        re = jnp.dot(fre[...], hb, preferred_element_type=F32)
        im = jnp.dot(fim[...], hb, preferred_element_type=F32)
        amp = jnp.sqrt(re * re + im * im).mean(axis=1).reshape(1, NUM_FREQS)
        logits = jnp.dot(amp, wg[...], preferred_element_type=F32)
        E = logits.shape[1]
        rows.append(jnp.concatenate(
            [logits, jnp.full((1, 16 - E), -1e30, F32)], axis=1))
    out_ref[...] = jnp.concatenate(rows, axis=0).reshape(SB, 1, 16)


def _sc_bcast(v, idx):
    dn = lax.GatherDimensionNumbers(offset_dims=(), collapsed_slice_dims=(0,),
                                    start_index_map=(0,))
    return lax.gather(v, idx[:, None], dn, slice_sizes=(1,),
                      mode=lax.GatherScatterMode.PROMISE_IN_BOUNDS)


def _route(logits16):
    B = logits16.shape[0]
    info = plsc.get_sparse_core_info()
    NC = info.num_cores
    NW = NC * info.num_subcores
    mesh = plsc.VectorSubcoreMesh(core_axis_name="c", subcore_axis_name="s")
    lg_pad = jnp.concatenate(
        [logits16, jnp.full((NW - B, 16), -1e30, F32)], axis=0)

    @functools.partial(
        pl.kernel, mesh=mesh,
        out_type=[jax.ShapeDtypeStruct((NW, 16), F32),
                  jax.ShapeDtypeStruct((NW, 16), jnp.int32)],
        scratch_types=[pltpu.VMEM((16,), F32),
                       pltpu.VMEM((16,), F32),
                       pltpu.VMEM((16,), jnp.int32)],
        compiler_params=pltpu.CompilerParams(needs_layout_passes=False),
    )
    def route(lg_hbm, gates_hbm, idx_hbm, lg_v, gt_v, ix_v):
        wid = lax.axis_index("s") * NC + lax.axis_index("c")
        pltpu.sync_copy(lg_hbm.at[wid], lg_v)
        lv = lg_v[...]
        iota = lax.broadcasted_iota(jnp.int32, (16,), 0)
        ks, vs = plsc.sort_key_val(lv, iota, descending=True)
        zeros = iota * 0
        ones = zeros + 1
        m1v = _sc_bcast(ks, zeros)
        m2v = _sc_bcast(ks, ones)
        i1v = _sc_bcast(vs, zeros)
        i2v = _sc_bcast(vs, ones)
        rv = jnp.exp(m2v - m1v)
        gt = jnp.where(iota == i1v, 1.0,
                       jnp.where(iota == i2v, rv, 0.0)) / (1.0 + rv)
        gt_v[...] = gt
        ix_v[...] = jnp.where(iota == 0, i1v,
                              jnp.where(iota == 1, i2v, 0))
        pltpu.sync_copy(gt_v, gates_hbm.at[wid])
        pltpu.sync_copy(ix_v, idx_hbm.at[wid])

    gates32, idx32 = route(lg_pad)
    return gates32[:B], idx32[:B]


def _expert_body(*refs, B, HW, ND):
    idx_ref = refs[0]
    gates_ref = refs[1]
    x_ref = refs[2]
    acc_ref = refs[-1]
    o_ref = refs[-2]
    wrefs = refs[3:-2]
    b = pl.program_id(0)
    kk = pl.program_id(1)
    e = idx_ref[b, kk]

    @pl.when(kk == 0)
    def _():
        acc_ref[...] = jnp.zeros_like(acc_ref)

    xv = x_ref[...]
    h_in = jnp.concatenate(
        [xv[t * HW:(t + 1) * HW, :].T for t in range(SEQ_LEN)], axis=0)

    g = gates_ref[b, e]
    for i, p in enumerate(SEGMENT_SIZES):
        n = SEQ_LEN // p
        ind = p * D_MODEL
        hid = (p // 2) * D_MODEL
        hd = hid // N_HEADS
        wq, wk, wv, wo, bo, peT = wrefs[6 * i:6 * i + 6]

        @pl.when((e == i) & (g > 0.0))
        def _(wq=wq, wk=wk, wv=wv, wo=wo, bo=bo, peT=peT,
              n=n, ind=ind, hid=hid, hd=hd):
            h = h_in
            pe = peT[...]
            qs, ks, vs = [], [], []
            for s in range(n):
                hs = h[s * ind:(s + 1) * ind, :]
                qs.append(jnp.dot(wq[...], hs, preferred_element_type=F32)
                          + pe[:, s:s + 1])
                ks.append(jnp.dot(wk[...], hs, preferred_element_type=F32)
                          + pe[:, n + s:n + s + 1])
                vs.append(jnp.dot(wv[...], hs, preferred_element_type=F32)
                          + pe[:, 2 * n + s:2 * n + s + 1])
            scale = 1.0 / math.sqrt(hd)
            vg = [v.reshape(N_HEADS, hd, HW) for v in vs]
            for s in range(n):
                sc = []
                for t in range(n):
                    prod = (qs[s] * ks[t]).reshape(N_HEADS, hd, HW)
                    sc.append(prod.sum(axis=1) * scale)
                m = sc[0]
                for t in range(1, n):
                    m = jnp.maximum(m, sc[t])
                ex = [jnp.exp(x - m) for x in sc]
                z = ex[0]
                for t in range(1, n):
                    z = z + ex[t]
                inv = 1.0 / z
                acc = (ex[0] * inv)[:, None, :] * vg[0]
                for t in range(1, n):
                    acc = acc + (ex[t] * inv)[:, None, :] * vg[t]
                out_s = acc.reshape(hid, HW)
                o_s = jnp.dot(wo[...], out_s,
                              preferred_element_type=F32) + bo[...]
                acc_ref[s * ind:(s + 1) * ind, :] += jnp.exp(o_s) * g

    @pl.when(kk == TOP_K - 1)
    def _():
        c = acc_ref[...]
        c = jnp.log(jnp.where(c == 0.0, EPS, c))
        for t in range(SEQ_LEN):
            o_ref[t * HW:(t + 1) * HW, :] = (
                c[t * D_MODEL:(t + 1) * D_MODEL, :].T)


def kernel(x, params):
    B, T, H, W, D = x.shape
    HW = H * W
    ND = T * D
    nE = len(SEGMENT_SIZES)

    t = np.arange(SEQ_LEN, dtype=np.float64)
    f = np.arange(1, NUM_FREQS + 1, dtype=np.float64)
    ang = 2.0 * np.pi * f[:, None] * t[None, :] / SEQ_LEN
    scale = 1.0 / np.sqrt(SEQ_LEN)
    fre = jnp.asarray((np.cos(ang) * scale).astype(np.float32))
    fim = jnp.asarray((-np.sin(ang) * scale).astype(np.float32))
    SB = 2
    gargs = [x.reshape(B * T * H * W, D)]
    gspecs = [pl.BlockSpec((SB * T * H * W, D), lambda b: (b, 0))]
    gfull = lambda b: (0, 0)
    for c in params['convs']:
        cin = c['w'].shape[1]
        cout = c['w'].shape[0]
        gargs += [c['w'].transpose(2, 3, 1, 0).reshape(4 * cin, cout),
                  c['b'][None], c['g'][None], c['beta'][None]]
        gspecs += [pl.BlockSpec((4 * cin, cout), gfull),
                   pl.BlockSpec((1, cout), gfull),
                   pl.BlockSpec((1, cout), gfull),
                   pl.BlockSpec((1, cout), gfull)]
    gargs += [params['fuse_w'], params['fuse_b'][None], fre, fim,
              params['w_gate']]
    gspecs += [pl.BlockSpec(params['fuse_w'].shape, gfull),
               pl.BlockSpec((1, params['fuse_b'].shape[0]), gfull),
               pl.BlockSpec(fre.shape, gfull),
               pl.BlockSpec(fim.shape, gfull),
               pl.BlockSpec(params['w_gate'].shape, gfull)]
    logits16 = pl.pallas_call(
        functools.partial(_gate_body, SB=SB),
        grid=(B // SB,),
        in_specs=gspecs,
        out_specs=pl.BlockSpec((SB, 1, 16), lambda b: (b, 0, 0)),
        out_shape=jax.ShapeDtypeStruct((B, 1, 16), F32),
        compiler_params=pltpu.CompilerParams(
            dimension_semantics=("arbitrary",),
            vmem_limit_bytes=100 * 1024 * 1024),
    )(*gargs).reshape(B, 16)
    gates16, idx16 = _route(logits16)
    gates = gates16[:, :nE]
    idx2 = idx16[:, :TOP_K]

    xv = x.reshape(B * T * HW, D)

    in_specs = [
        pl.BlockSpec(memory_space=pltpu.SMEM),
        pl.BlockSpec((T * HW, D), lambda b, k, idx: (b, 0)),
    ]
    args = [gates, xv]
    full = lambda b, k, idx: (0, 0)
    for ei, p in enumerate(SEGMENT_SIZES):
        ep = params['experts'][ei]
        n = SEQ_LEN // p
        ind = p * D_MODEL
        hid = (p // 2) * D_MODEL
        pe = jnp.asarray(_sin_pe_np(n, ind))
        peT = jnp.concatenate(
            [jnp.dot(ep['wq'], pe.T), jnp.dot(ep['wk'], pe.T),
             jnp.dot(ep['wv'], pe.T)], axis=1)
        for a in (ep['wq'], ep['wk'], ep['wv'], ep['wo']):
            args.append(a)
            in_specs.append(pl.BlockSpec(a.shape, full))
        args.append(ep['bo'][:, None])
        in_specs.append(pl.BlockSpec((ind, 1), full))
        args.append(peT)
        in_specs.append(pl.BlockSpec((hid, 3 * n), full))

    out = pl.pallas_call(
        functools.partial(_expert_body, B=B, HW=HW, ND=ND),
        grid_spec=pltpu.PrefetchScalarGridSpec(
            num_scalar_prefetch=1,
            grid=(B, TOP_K),
            in_specs=in_specs,
            out_specs=pl.BlockSpec((T * HW, D), lambda b, k, idx: (b, 0)),
            scratch_shapes=[pltpu.VMEM((ND, HW), F32)],
        ),
        out_shape=jax.ShapeDtypeStruct((B * T * HW, D), F32),
        compiler_params=pltpu.CompilerParams(
            dimension_semantics=("arbitrary", "arbitrary"),
            vmem_limit_bytes=100 * 1024 * 1024,
        ),
    )(idx2, *args)

    return out.reshape(B, T, H, W, D)
